# baseline (speedup 1.0000x reference)
import functools

import jax
import jax.numpy as jnp
from jax import lax
from jax.experimental import pallas as pl
from jax.experimental.pallas import tpu as pltpu
from jax.experimental.pallas import tpu_sc as plsc

EMB = 64
SEQ = 200
LANES = 16
NC = 2
NS = 16
NW = NC * NS

SUB = 100
NSUB = 8
CHUNK = SUB * NSUB


def _sc_embed(table, x2d, pos_enc, n_tok):
    rows_per_w = n_tok // NW
    n_chunks = rows_per_w // CHUNK
    mesh = plsc.VectorSubcoreMesh(core_axis_name="c", subcore_axis_name="s")

    @functools.partial(
        pl.kernel,
        out_type=jax.ShapeDtypeStruct((n_tok, EMB), jnp.float32),
        mesh=mesh,
        compiler_params=pltpu.CompilerParams(use_tc_tiling_on_sc=False),
        scratch_types=[
            pltpu.VMEM((NSUB, SUB), jnp.int32),
            pltpu.VMEM((CHUNK, EMB), jnp.float32),
            pltpu.VMEM((SEQ, EMB), jnp.float32),
            pltpu.SemaphoreType.DMA,
        ],
    )
    def body(table_hbm, x_hbm, pe_hbm, out_hbm, idx_v, rows_v, pe_v, sem):
        wid = lax.axis_index("s") * NC + lax.axis_index("c")
        base_w = wid * rows_per_w
        pltpu.sync_copy(pe_hbm, pe_v)

        def chunk_body(c, carry):
            base = base_w + c * CHUNK
            pltpu.sync_copy(
                x_hbm.at[pl.ds(pl.multiple_of(base // SUB, NSUB), NSUB)], idx_v
            )
            cps = [
                pltpu.async_copy(
                    table_hbm.at[idx_v.at[j]],
                    rows_v.at[pl.ds(j * SUB, SUB)],
                    sem,
                )
                for j in range(NSUB)
            ]
            for cp in cps:
                cp.wait()

            def pos_body(s, carry2):
                for d in range(EMB // LANES):
                    pv = pe_v[s, pl.ds(d * LANES, LANES)]
                    for b in range(CHUNK // SEQ):
                        r = s + b * SEQ
                        rows_v[r, pl.ds(d * LANES, LANES)] = (
                            rows_v[r, pl.ds(d * LANES, LANES)] + pv
                        )
                return carry2

            lax.fori_loop(0, SEQ, pos_body, 0)
            pltpu.sync_copy(rows_v, out_hbm.at[pl.ds(base, CHUNK)])
            return carry

        lax.fori_loop(0, n_chunks, chunk_body, 0)

    return body(table, x2d, pos_enc)


def kernel(x, table, pos_enc):
    batch, seq = x.shape
    n_tok = batch * seq
    x2d = x.reshape(n_tok // SUB, SUB)
    out = _sc_embed(table, x2d, pos_enc[:seq], n_tok)
    return out.reshape(batch, seq, EMB)
